# Initial kernel scaffold; baseline (speedup 1.0000x reference)
import jax
import jax.numpy as jnp
from jax.experimental import pallas as pl


def kernel(x, edge_index, batch, W0, as0, ad0, b0, W1, as1, ad1, b1, W2, as2, ad2, b2, bnw0, bnb0, bnw1, bnb1, Wp1, bp1, Wp2, bp2):
    raise NotImplementedError("write your pallas kernel here")


# speedup vs baseline: 31.5775x; 31.5775x over previous
import functools

import jax
import jax.numpy as jnp
from jax import lax
from jax.experimental import pallas as pl
from jax.experimental.pallas import tpu as pltpu
from jax.experimental.pallas import tpu_sc as plsc

N = 10000
NP = 10240
E = 320000
E2 = E + N
CHUNK = 128
NCHUNK = 2592
EP = NCHUNK * CHUNK
ROWS_PER_TILE = NP // 16
G = 64
NEG_INF = float("-inf")


def _tc_a0_body(x_ref, w_ref, as_ref, ad_ref, hp_ref, als_ref, ald_ref):
  h = jnp.dot(x_ref[...], w_ref[...], preferred_element_type=jnp.float32)
  hp_ref[0] = h[:, :128]
  hp_ref[1] = h[:, 128:]
  als_ref[...] = jnp.dot(h, as_ref[...], preferred_element_type=jnp.float32)
  ald_ref[...] = jnp.dot(h, ad_ref[...], preferred_element_type=jnp.float32)


def _tc_a0(x_pad, W0, As, Ad):
  blk = NP // 8
  return pl.pallas_call(
      _tc_a0_body,
      grid=(8,),
      in_specs=[
          pl.BlockSpec((blk, 128), lambda i: (i, 0)),
          pl.BlockSpec((128, 256), lambda i: (0, 0)),
          pl.BlockSpec((256, 4), lambda i: (0, 0)),
          pl.BlockSpec((256, 4), lambda i: (0, 0)),
      ],
      out_specs=[
          pl.BlockSpec((2, blk, 128), lambda i: (0, i, 0)),
          pl.BlockSpec((blk, 4), lambda i: (i, 0)),
          pl.BlockSpec((blk, 4), lambda i: (i, 0)),
      ],
      out_shape=[
          jax.ShapeDtypeStruct((2, NP, 128), jnp.float32),
          jax.ShapeDtypeStruct((NP, 4), jnp.float32),
          jax.ShapeDtypeStruct((NP, 4), jnp.float32),
      ],
  )(x_pad, W0, As, Ad)


def _tc_mid_body(acc_ref, den_ref, b_ref, sc_ref, sh_ref, w_ref, as_ref,
                 ad_ref, hp_ref, als_ref, ald_ref, *, nheads_next):
  a0 = acc_ref[0]
  a1 = acc_ref[1]
  d0 = den_ref[0]
  d1 = den_ref[1]
  eps = jnp.float32(1e-16)
  p0 = a0[:, :64] / (d0[:, 0:1] + eps)
  p1 = a0[:, 64:] / (d0[:, 1:2] + eps)
  p2 = a1[:, :64] / (d1[:, 0:1] + eps)
  p3 = a1[:, 64:] / (d1[:, 1:2] + eps)
  xx = jnp.concatenate([p0, p1, p2, p3], axis=1) + b_ref[...]
  xx = xx * sc_ref[...] + sh_ref[...]
  xx = jnp.where(xx > 0, xx, jnp.exp(xx) - 1.0)
  h = jnp.dot(xx, w_ref[...], preferred_element_type=jnp.float32)
  if nheads_next == 4:
    hp_ref[0] = h[:, :128]
    hp_ref[1] = h[:, 128:]
  else:
    hp_ref[...] = h
  als_ref[...] = jnp.dot(h, as_ref[...], preferred_element_type=jnp.float32)
  ald_ref[...] = jnp.dot(h, ad_ref[...], preferred_element_type=jnp.float32)


def _tc_mid(acc, den, b_row, sc_row, sh_row, W, As, Ad, nheads_next):
  blk = NP // 8
  dout = W.shape[1]
  nh = nheads_next
  if nh == 4:
    hp_spec = pl.BlockSpec((2, blk, 128), lambda i: (0, i, 0))
    hp_shape = jax.ShapeDtypeStruct((2, NP, 128), jnp.float32)
  else:
    hp_spec = pl.BlockSpec((blk, 64), lambda i: (i, 0))
    hp_shape = jax.ShapeDtypeStruct((NP, 64), jnp.float32)
  return pl.pallas_call(
      functools.partial(_tc_mid_body, nheads_next=nh),
      grid=(8,),
      in_specs=[
          pl.BlockSpec((2, blk, 128), lambda i: (0, i, 0)),
          pl.BlockSpec((2, blk, 16), lambda i: (0, i, 0)),
          pl.BlockSpec((1, 256), lambda i: (0, 0)),
          pl.BlockSpec((1, 256), lambda i: (0, 0)),
          pl.BlockSpec((1, 256), lambda i: (0, 0)),
          pl.BlockSpec((256, dout), lambda i: (0, 0)),
          pl.BlockSpec((dout, nh), lambda i: (0, 0)),
          pl.BlockSpec((dout, nh), lambda i: (0, 0)),
      ],
      out_specs=[
          hp_spec,
          pl.BlockSpec((blk, nh), lambda i: (i, 0)),
          pl.BlockSpec((blk, nh), lambda i: (i, 0)),
      ],
      out_shape=[
          hp_shape,
          jax.ShapeDtypeStruct((NP, nh), jnp.float32),
          jax.ShapeDtypeStruct((NP, nh), jnp.float32),
      ],
  )(acc, den, b_row, sc_row, sh_row, W, As, Ad)


def _tc_final_body(acc_ref, den_ref, b2_ref, batch_ref, wp1_ref, bp1_ref,
                   wp2_ref, bp2_ref, out_ref, xm_ref):
  eps = jnp.float32(1e-16)
  den = den_ref[0][:, 0:1] + den_ref[1][:, 0:1] + eps
  x3 = (acc_ref[0] + acc_ref[1]) / den + b2_ref[...]
  bcol = batch_ref[...]
  gids = lax.broadcasted_iota(jnp.int32, (1, G), 1)
  S = (bcol == gids).astype(jnp.float32)
  counts = jnp.sum(S, axis=0)
  xsum = lax.dot_general(S, x3, (((0,), (0,)), ((), ())),
                         preferred_element_type=jnp.float32)
  xmean = xsum / jnp.maximum(counts, 1.0)[:, None]
  def _seg_max(g, _):
    mg = jnp.where(bcol == g, x3, NEG_INF)
    xm_ref[pl.ds(g, 1), :] = jnp.max(mg, axis=0)[None, :]
    return 0
  lax.fori_loop(0, G, _seg_max, 0)
  xmax = jnp.where(counts[:, None] > 0, xm_ref[...], 0.0)
  xcat = jnp.concatenate([xmax, xmean], axis=1)
  h1 = jnp.dot(xcat, wp1_ref[...], preferred_element_type=jnp.float32)
  h1 = jnp.maximum(h1 + bp1_ref[...], 0.0)
  out_ref[...] = jnp.dot(h1, wp2_ref[...],
                         preferred_element_type=jnp.float32) + bp2_ref[...]


def _tc_final(acc2, den2, b2_row, batch_col, Wp1, bp1_row, Wp2, bp2_row):
  return pl.pallas_call(
      _tc_final_body,
      out_shape=jax.ShapeDtypeStruct((G, 64), jnp.float32),
      scratch_shapes=[pltpu.VMEM((G, 64), jnp.float32)],
  )(acc2, den2, b2_row, batch_col, Wp1, bp1_row, Wp2, bp2_row)


_MESH = plsc.VectorSubcoreMesh(core_axis_name="c", subcore_axis_name="s")
_SC_PARAMS = pltpu.CompilerParams(needs_layout_passes=False,
                                  use_tc_tiling_on_sc=False)


@functools.partial(
    pl.kernel,
    out_type=(jax.ShapeDtypeStruct((2, 2 * EP), jnp.float32),
              jax.ShapeDtypeStruct((2, NP, 16), jnp.float32)),
    mesh=_MESH,
    compiler_params=_SC_PARAMS,
    scratch_types=(
        pltpu.VMEM((2 * NP,), jnp.float32),
        pltpu.VMEM((2 * NP,), jnp.float32),
        pltpu.VMEM((CHUNK,), jnp.int32),
        pltpu.VMEM((1, CHUNK), jnp.int32),
        pltpu.VMEM((2 * CHUNK,), jnp.float32),
        pltpu.VMEM((CHUNK, 16), jnp.float32),
        pltpu.VMEM_SHARED((NP, 16), jnp.float32),
        pltpu.SemaphoreType.DMA,
    ),
)
def _sc_w4(src_hbm, dst_hbm, als_hbm, ald_hbm, w_out, den_out,
           als_l, ald_l, srcb, dstb, woutb, wvec, den_sp, sem):
  c = lax.axis_index("c")
  s = lax.axis_index("s")
  pltpu.sync_copy(als_hbm.at[pl.ds(c * 2 * NP, 2 * NP)], als_l)
  pltpu.sync_copy(ald_hbm.at[pl.ds(c * 2 * NP, 2 * NP)], ald_l)

  zero16 = jnp.zeros((16,), jnp.float32)

  @pl.loop(0, CHUNK)
  def _(i):
    wvec[i, :] = zero16

  row0 = s * ROWS_PER_TILE
  for z in range(ROWS_PER_TILE // CHUNK):
    pltpu.sync_copy(wvec, den_sp.at[pl.ds(row0 + z * CHUNK, CHUNK), :])
  plsc.subcore_barrier()

  iota16 = lax.broadcasted_iota(jnp.int32, (16,), 0)
  per_tile = NCHUNK // 16

  @pl.loop(0, per_tile)
  def _(k):
    base = (s * per_tile + k) * CHUNK
    pltpu.sync_copy(src_hbm.at[pl.ds(base, CHUNK)], srcb)
    pltpu.sync_copy(dst_hbm.at[pl.ds(base, CHUNK)], dstb.at[0])
    for j in range(8):
      s16 = srcb[pl.ds(j * 16, 16)]
      d16 = dstb[0, pl.ds(j * 16, 16)]
      e16 = j * 16 + iota16
      for hd in range(2):
        av = plsc.load_gather(als_l, [s16 * 2 + hd])
        dv = plsc.load_gather(ald_l, [d16 * 2 + hd])
        t = av + dv
        w = jnp.exp(jnp.where(t >= 0, t, 0.2 * t))
        woutb[pl.ds(hd * CHUNK + j * 16, 16)] = w
        plsc.store_scatter(wvec, [e16, jnp.full((16,), hd, jnp.int32)], w)
    pltpu.sync_copy(wvec, den_sp.at[dstb.at[0]], add=True)
    pltpu.sync_copy(woutb.at[pl.ds(0, CHUNK)], w_out.at[c, pl.ds(base, CHUNK)])
    pltpu.sync_copy(woutb.at[pl.ds(CHUNK, CHUNK)],
                    w_out.at[c, pl.ds(EP + base, CHUNK)])

  plsc.subcore_barrier()
  for z in range(ROWS_PER_TILE // CHUNK):
    rows = pl.ds(row0 + z * CHUNK, CHUNK)
    pltpu.sync_copy(den_sp.at[rows, :], wvec)
    pltpu.sync_copy(wvec, den_out.at[c, rows, :])


@functools.partial(
    pl.kernel,
    out_type=jax.ShapeDtypeStruct((2, NP, 128), jnp.float32),
    mesh=_MESH,
    compiler_params=_SC_PARAMS,
    scratch_types=(
        pltpu.VMEM((CHUNK,), jnp.int32),
        pltpu.VMEM((CHUNK,), jnp.int32),
        pltpu.VMEM((1, CHUNK), jnp.int32),
        pltpu.VMEM((CHUNK,), jnp.float32),
        pltpu.VMEM((CHUNK,), jnp.float32),
        pltpu.VMEM((CHUNK, 128), jnp.float32),
        pltpu.VMEM_SHARED((NP, 128), jnp.float32),
        pltpu.SemaphoreType.DMA,
    ),
)
def _sc_f4(src_hbm, dst_hbm, h_hbm, w_hbm, acc_out,
           srcb, gidx, dstb, wb0, wb1, rowb, acc_sp, sem):
  c = lax.axis_index("c")
  s = lax.axis_index("s")

  zero16 = jnp.zeros((16,), jnp.float32)

  @pl.loop(0, CHUNK)
  def _(i):
    for j in range(8):
      rowb[i, pl.ds(j * 16, 16)] = zero16

  row0 = s * ROWS_PER_TILE
  for z in range(ROWS_PER_TILE // CHUNK):
    pltpu.sync_copy(rowb, acc_sp.at[pl.ds(row0 + z * CHUNK, CHUNK), :])
  plsc.subcore_barrier()

  coff = c * NP
  per_tile = NCHUNK // 16

  @pl.loop(0, per_tile)
  def _(k):
    base = (s * per_tile + k) * CHUNK
    pltpu.sync_copy(src_hbm.at[pl.ds(base, CHUNK)], srcb)
    for j in range(8):
      gidx[pl.ds(j * 16, 16)] = srcb[pl.ds(j * 16, 16)] + coff
    cp = pltpu.async_copy(h_hbm.at[gidx], rowb, sem)
    pltpu.sync_copy(dst_hbm.at[pl.ds(base, CHUNK)], dstb.at[0])
    pltpu.sync_copy(w_hbm.at[c, pl.ds(base, CHUNK)], wb0)
    pltpu.sync_copy(w_hbm.at[c, pl.ds(EP + base, CHUNK)], wb1)
    cp.wait()

    @pl.loop(0, CHUNK)
    def _(i):
      i16 = jnp.full((16,), i, jnp.int32)
      w0 = plsc.load_gather(wb0, [i16])
      w1 = plsc.load_gather(wb1, [i16])
      for j in range(8):
        v = rowb[i, pl.ds(j * 16, 16)]
        rowb[i, pl.ds(j * 16, 16)] = v * (w0 if j < 4 else w1)

    pltpu.sync_copy(rowb, acc_sp.at[dstb.at[0]], add=True)

  plsc.subcore_barrier()
  for z in range(ROWS_PER_TILE // CHUNK):
    rows = pl.ds(row0 + z * CHUNK, CHUNK)
    pltpu.sync_copy(acc_sp.at[rows, :], rowb)
    pltpu.sync_copy(rowb, acc_out.at[c, rows, :])


@functools.partial(
    pl.kernel,
    out_type=(jax.ShapeDtypeStruct((2, NP, 64), jnp.float32),
              jax.ShapeDtypeStruct((2, NP, 16), jnp.float32)),
    mesh=_MESH,
    compiler_params=_SC_PARAMS,
    scratch_types=(
        pltpu.VMEM((NP,), jnp.float32),
        pltpu.VMEM((NP,), jnp.float32),
        pltpu.VMEM((CHUNK,), jnp.int32),
        pltpu.VMEM((1, CHUNK), jnp.int32),
        pltpu.VMEM((CHUNK, 64), jnp.float32),
        pltpu.VMEM((CHUNK, 16), jnp.float32),
        pltpu.VMEM((CHUNK,), jnp.float32),
        pltpu.VMEM_SHARED((NP, 64), jnp.float32),
        pltpu.VMEM_SHARED((NP, 16), jnp.float32),
        pltpu.SemaphoreType.DMA,
    ),
)
def _sc_edge1(src_hbm, dst_hbm, h_hbm, als_hbm, ald_hbm, acc_out, den_out,
              als_l, ald_l, srcb, dstb, rowb, wvec, wb0, acc_sp, den_sp, sem):
  c = lax.axis_index("c")
  s = lax.axis_index("s")
  pltpu.sync_copy(als_hbm, als_l)
  pltpu.sync_copy(ald_hbm, ald_l)

  zero16 = jnp.zeros((16,), jnp.float32)

  @pl.loop(0, CHUNK)
  def _(i):
    for j in range(4):
      rowb[i, pl.ds(j * 16, 16)] = zero16
    wvec[i, :] = zero16

  row0 = s * ROWS_PER_TILE
  for z in range(ROWS_PER_TILE // CHUNK):
    pltpu.sync_copy(rowb, acc_sp.at[pl.ds(row0 + z * CHUNK, CHUNK), :])
    pltpu.sync_copy(wvec, den_sp.at[pl.ds(row0 + z * CHUNK, CHUNK), :])
  plsc.subcore_barrier()

  iota16 = lax.broadcasted_iota(jnp.int32, (16,), 0)
  per_tile = NCHUNK // 32

  @pl.loop(0, per_tile)
  def _(k):
    base = (c * (NCHUNK // 2) + s * per_tile + k) * CHUNK
    pltpu.sync_copy(src_hbm.at[pl.ds(base, CHUNK)], srcb)
    pltpu.sync_copy(dst_hbm.at[pl.ds(base, CHUNK)], dstb.at[0])
    cp = pltpu.async_copy(h_hbm.at[srcb], rowb, sem)
    for j in range(8):
      s16 = srcb[pl.ds(j * 16, 16)]
      d16 = dstb[0, pl.ds(j * 16, 16)]
      e16 = j * 16 + iota16
      av = plsc.load_gather(als_l, [s16])
      dv = plsc.load_gather(ald_l, [d16])
      t = av + dv
      w = jnp.exp(jnp.where(t >= 0, t, 0.2 * t))
      wb0[pl.ds(j * 16, 16)] = w
      plsc.store_scatter(wvec, [e16, jnp.zeros((16,), jnp.int32)], w)
    cp.wait()

    @pl.loop(0, CHUNK)
    def _(i):
      i16 = jnp.full((16,), i, jnp.int32)
      w0 = plsc.load_gather(wb0, [i16])
      for j in range(4):
        v = rowb[i, pl.ds(j * 16, 16)]
        rowb[i, pl.ds(j * 16, 16)] = v * w0

    pltpu.sync_copy(rowb, acc_sp.at[dstb.at[0]], add=True)
    pltpu.sync_copy(wvec, den_sp.at[dstb.at[0]], add=True)

  plsc.subcore_barrier()
  for z in range(ROWS_PER_TILE // CHUNK):
    rows = pl.ds(row0 + z * CHUNK, CHUNK)
    pltpu.sync_copy(acc_sp.at[rows, :], rowb)
    pltpu.sync_copy(rowb, acc_out.at[c, rows, :])
    pltpu.sync_copy(den_sp.at[rows, :], wvec)
    pltpu.sync_copy(wvec, den_out.at[c, rows, :])


def _block_diag_att(a):
  h, cdim = a.shape
  m = jnp.zeros((h * cdim, h), jnp.float32)
  for i in range(h):
    m = m.at[i * cdim:(i + 1) * cdim, i].set(a[i])
  return m


def kernel(x, edge_index, batch, W0, as0, ad0, b0, W1, as1, ad1, b1,
           W2, as2, ad2, b2, bnw0, bnb0, bnw1, bnb1, Wp1, bp1, Wp2, bp2):
  npad = EP - E2
  pad_idx = (N + (jnp.arange(npad, dtype=jnp.int32) % (NP - N))).astype(jnp.int32)
  loop_idx = jnp.arange(N, dtype=jnp.int32)
  src = jnp.concatenate([edge_index[0].astype(jnp.int32), loop_idx, pad_idx])
  dst = jnp.concatenate([edge_index[1].astype(jnp.int32), loop_idx, pad_idx])
  x_pad = jnp.pad(x, ((0, NP - N), (0, 0)))
  batch_col = jnp.pad(batch.astype(jnp.int32), (0, NP - N),
                      constant_values=G).reshape(NP, 1)

  As0 = _block_diag_att(as0)
  Ad0 = _block_diag_att(ad0)
  As1 = _block_diag_att(as1)
  Ad1 = _block_diag_att(ad1)
  As2 = as2.reshape(64, 1)
  Ad2 = ad2.reshape(64, 1)

  inv = jnp.float32(1.0 / jnp.sqrt(1.0 + 1e-5))
  b0r = b0.reshape(1, 256)
  sc0 = (bnw0 * inv).reshape(1, 256)
  sh0 = bnb0.reshape(1, 256)
  b1r = b1.reshape(1, 256)
  sc1 = (bnw1 * inv).reshape(1, 256)
  sh1 = bnb1.reshape(1, 256)
  b2r = b2.reshape(1, 64)

  hp0, als0v, ald0v = _tc_a0(x_pad, W0, As0, Ad0)
  h0flat = hp0.reshape(2 * NP, 128)
  als0p = als0v.reshape(NP, 2, 2).transpose(1, 0, 2).reshape(4 * NP)
  ald0p = ald0v.reshape(NP, 2, 2).transpose(1, 0, 2).reshape(4 * NP)
  w0e, den0 = _sc_w4(src, dst, als0p, ald0p)
  acc0 = _sc_f4(src, dst, h0flat, w0e)

  hp1, als1v, ald1v = _tc_mid(acc0, den0, b0r, sc0, sh0, W1, As1, Ad1, 4)
  h1flat = hp1.reshape(2 * NP, 128)
  als1p = als1v.reshape(NP, 2, 2).transpose(1, 0, 2).reshape(4 * NP)
  ald1p = ald1v.reshape(NP, 2, 2).transpose(1, 0, 2).reshape(4 * NP)
  w1e, den1 = _sc_w4(src, dst, als1p, ald1p)
  acc1 = _sc_f4(src, dst, h1flat, w1e)

  h2, als2v, ald2v = _tc_mid(acc1, den1, b1r, sc1, sh1, W2, As2, Ad2, 1)
  acc2, den2 = _sc_edge1(src, dst, h2, als2v.reshape(NP), ald2v.reshape(NP))

  return _tc_final(acc2, den2, b2r, batch_col, Wp1, bp1.reshape(1, 64),
                   Wp2, bp2.reshape(1, 64))


# speedup vs baseline: 41.8261x; 1.3246x over previous
import functools

import jax
import jax.numpy as jnp
from jax import lax
from jax.experimental import pallas as pl
from jax.experimental.pallas import tpu as pltpu
from jax.experimental.pallas import tpu_sc as plsc

N = 10000
NP = 10240
E = 320000
E2 = E + N
CHUNK = 128
NCHUNK = 2592
EP = NCHUNK * CHUNK
ROWS_PER_TILE = NP // 16
G = 64
NEG_INF = float("-inf")


def _tc_a0_body(x_ref, w_ref, as_ref, ad_ref, hp_ref, als_ref, ald_ref):
  h = jnp.dot(x_ref[...], w_ref[...], preferred_element_type=jnp.float32)
  hp_ref[0] = h[:, :128]
  hp_ref[1] = h[:, 128:]
  als_ref[...] = jnp.dot(h, as_ref[...], preferred_element_type=jnp.float32)
  ald_ref[...] = jnp.dot(h, ad_ref[...], preferred_element_type=jnp.float32)


def _tc_a0(x_pad, W0, As, Ad):
  blk = NP // 8
  return pl.pallas_call(
      _tc_a0_body,
      grid=(8,),
      in_specs=[
          pl.BlockSpec((blk, 128), lambda i: (i, 0)),
          pl.BlockSpec((128, 256), lambda i: (0, 0)),
          pl.BlockSpec((256, 4), lambda i: (0, 0)),
          pl.BlockSpec((256, 4), lambda i: (0, 0)),
      ],
      out_specs=[
          pl.BlockSpec((2, blk, 128), lambda i: (0, i, 0)),
          pl.BlockSpec((blk, 4), lambda i: (i, 0)),
          pl.BlockSpec((blk, 4), lambda i: (i, 0)),
      ],
      out_shape=[
          jax.ShapeDtypeStruct((2, NP, 128), jnp.float32),
          jax.ShapeDtypeStruct((NP, 4), jnp.float32),
          jax.ShapeDtypeStruct((NP, 4), jnp.float32),
      ],
  )(x_pad, W0, As, Ad)


def _tc_mid_body(acc_ref, den_ref, b_ref, sc_ref, sh_ref, w_ref, as_ref,
                 ad_ref, hp_ref, als_ref, ald_ref, *, nheads_next):
  a0 = acc_ref[0]
  a1 = acc_ref[1]
  d0 = den_ref[0]
  d1 = den_ref[1]
  eps = jnp.float32(1e-16)
  p0 = a0[:, :64] / (d0[:, 0:1] + eps)
  p1 = a0[:, 64:] / (d0[:, 1:2] + eps)
  p2 = a1[:, :64] / (d1[:, 0:1] + eps)
  p3 = a1[:, 64:] / (d1[:, 1:2] + eps)
  xx = jnp.concatenate([p0, p1, p2, p3], axis=1) + b_ref[...]
  xx = xx * sc_ref[...] + sh_ref[...]
  xx = jnp.where(xx > 0, xx, jnp.exp(xx) - 1.0)
  h = jnp.dot(xx, w_ref[...], preferred_element_type=jnp.float32)
  if nheads_next == 4:
    hp_ref[0] = h[:, :128]
    hp_ref[1] = h[:, 128:]
  else:
    hp_ref[...] = h
  als_ref[...] = jnp.dot(h, as_ref[...], preferred_element_type=jnp.float32)
  ald_ref[...] = jnp.dot(h, ad_ref[...], preferred_element_type=jnp.float32)


def _tc_mid(acc, den, b_row, sc_row, sh_row, W, As, Ad, nheads_next):
  blk = NP // 8
  dout = W.shape[1]
  nh = nheads_next
  if nh == 4:
    hp_spec = pl.BlockSpec((2, blk, 128), lambda i: (0, i, 0))
    hp_shape = jax.ShapeDtypeStruct((2, NP, 128), jnp.float32)
  else:
    hp_spec = pl.BlockSpec((blk, 64), lambda i: (i, 0))
    hp_shape = jax.ShapeDtypeStruct((NP, 64), jnp.float32)
  return pl.pallas_call(
      functools.partial(_tc_mid_body, nheads_next=nh),
      grid=(8,),
      in_specs=[
          pl.BlockSpec((2, blk, 128), lambda i: (0, i, 0)),
          pl.BlockSpec((2, blk, 16), lambda i: (0, i, 0)),
          pl.BlockSpec((1, 256), lambda i: (0, 0)),
          pl.BlockSpec((1, 256), lambda i: (0, 0)),
          pl.BlockSpec((1, 256), lambda i: (0, 0)),
          pl.BlockSpec((256, dout), lambda i: (0, 0)),
          pl.BlockSpec((dout, nh), lambda i: (0, 0)),
          pl.BlockSpec((dout, nh), lambda i: (0, 0)),
      ],
      out_specs=[
          hp_spec,
          pl.BlockSpec((blk, nh), lambda i: (i, 0)),
          pl.BlockSpec((blk, nh), lambda i: (i, 0)),
      ],
      out_shape=[
          hp_shape,
          jax.ShapeDtypeStruct((NP, nh), jnp.float32),
          jax.ShapeDtypeStruct((NP, nh), jnp.float32),
      ],
  )(acc, den, b_row, sc_row, sh_row, W, As, Ad)


def _tc_final_body(acc_ref, den_ref, b2_ref, batch_ref, wp1_ref, bp1_ref,
                   wp2_ref, bp2_ref, out_ref, xm_ref):
  eps = jnp.float32(1e-16)
  den = den_ref[0][:, 0:1] + den_ref[1][:, 0:1] + eps
  x3 = (acc_ref[0] + acc_ref[1]) / den + b2_ref[...]
  bcol = batch_ref[...]
  gids = lax.broadcasted_iota(jnp.int32, (1, G), 1)
  S = (bcol == gids).astype(jnp.float32)
  counts = jnp.sum(S, axis=0)
  xsum = lax.dot_general(S, x3, (((0,), (0,)), ((), ())),
                         preferred_element_type=jnp.float32)
  xmean = xsum / jnp.maximum(counts, 1.0)[:, None]
  def _seg_max(g, _):
    mg = jnp.where(bcol == g, x3, NEG_INF)
    xm_ref[pl.ds(g, 1), :] = jnp.max(mg, axis=0)[None, :]
    return 0
  lax.fori_loop(0, G, _seg_max, 0)
  xmax = jnp.where(counts[:, None] > 0, xm_ref[...], 0.0)
  xcat = jnp.concatenate([xmax, xmean], axis=1)
  h1 = jnp.dot(xcat, wp1_ref[...], preferred_element_type=jnp.float32)
  h1 = jnp.maximum(h1 + bp1_ref[...], 0.0)
  out_ref[...] = jnp.dot(h1, wp2_ref[...],
                         preferred_element_type=jnp.float32) + bp2_ref[...]


def _tc_final(acc2, den2, b2_row, batch_col, Wp1, bp1_row, Wp2, bp2_row):
  return pl.pallas_call(
      _tc_final_body,
      out_shape=jax.ShapeDtypeStruct((G, 64), jnp.float32),
      scratch_shapes=[pltpu.VMEM((G, 64), jnp.float32)],
  )(acc2, den2, b2_row, batch_col, Wp1, bp1_row, Wp2, bp2_row)


_MESH = plsc.VectorSubcoreMesh(core_axis_name="c", subcore_axis_name="s")
_SC_PARAMS = pltpu.CompilerParams(needs_layout_passes=False,
                                  use_tc_tiling_on_sc=False)


@functools.partial(
    pl.kernel,
    out_type=(jax.ShapeDtypeStruct((2, 2 * NCHUNK, CHUNK), jnp.float32),
              jax.ShapeDtypeStruct((2, NP, 16), jnp.float32)),
    mesh=_MESH,
    compiler_params=_SC_PARAMS,
    scratch_types=(
        pltpu.VMEM((2 * NP,), jnp.float32),
        pltpu.VMEM((2 * NP,), jnp.float32),
        pltpu.VMEM((CHUNK,), jnp.int32),
        pltpu.VMEM((1, CHUNK), jnp.int32),
        pltpu.VMEM((2 * CHUNK,), jnp.float32),
        pltpu.VMEM((CHUNK, 16), jnp.float32),
        pltpu.VMEM_SHARED((NP, 16), jnp.float32),
        pltpu.SemaphoreType.DMA,
    ),
)
def _sc_w4(src_hbm, dst_hbm, als_hbm, ald_hbm, w_out, den_out,
           als_l, ald_l, srcb, dstb, woutb, wvec, den_sp, sem):
  c = lax.axis_index("c")
  s = lax.axis_index("s")
  pltpu.sync_copy(als_hbm.at[pl.ds(c * 2 * NP, 2 * NP)], als_l)
  pltpu.sync_copy(ald_hbm.at[pl.ds(c * 2 * NP, 2 * NP)], ald_l)

  zero16 = jnp.zeros((16,), jnp.float32)

  @pl.loop(0, CHUNK)
  def _(i):
    wvec[i, :] = zero16

  row0 = s * ROWS_PER_TILE
  for z in range(ROWS_PER_TILE // CHUNK):
    pltpu.sync_copy(wvec, den_sp.at[pl.ds(row0 + z * CHUNK, CHUNK), :])
  plsc.subcore_barrier()

  iota16 = lax.broadcasted_iota(jnp.int32, (16,), 0)
  per_tile = NCHUNK // 16

  @pl.loop(0, per_tile)
  def _(k):
    base = (s * per_tile + k) * CHUNK
    pltpu.sync_copy(src_hbm.at[pl.ds(base, CHUNK)], srcb)
    pltpu.sync_copy(dst_hbm.at[pl.ds(base, CHUNK)], dstb.at[0])
    for j in range(8):
      s16 = srcb[pl.ds(j * 16, 16)]
      d16 = dstb[0, pl.ds(j * 16, 16)]
      e16 = j * 16 + iota16
      for hd in range(2):
        av = plsc.load_gather(als_l, [s16 * 2 + hd])
        dv = plsc.load_gather(ald_l, [d16 * 2 + hd])
        t = av + dv
        w = jnp.exp(jnp.where(t >= 0, t, 0.2 * t))
        woutb[pl.ds(hd * CHUNK + j * 16, 16)] = w
        plsc.store_scatter(wvec, [e16, jnp.full((16,), hd, jnp.int32)], w)
    pltpu.sync_copy(wvec, den_sp.at[dstb.at[0]], add=True)
    crow = s * per_tile + k
    pltpu.sync_copy(woutb.at[pl.ds(0, CHUNK)], w_out.at[c, crow, :])
    pltpu.sync_copy(woutb.at[pl.ds(CHUNK, CHUNK)],
                    w_out.at[c, NCHUNK + crow, :])

  plsc.subcore_barrier()
  for z in range(ROWS_PER_TILE // CHUNK):
    rows = pl.ds(row0 + z * CHUNK, CHUNK)
    pltpu.sync_copy(den_sp.at[rows, :], wvec)
    pltpu.sync_copy(wvec, den_out.at[c, rows, :])


@functools.partial(
    pl.kernel,
    out_type=jax.ShapeDtypeStruct((2, NP, 128), jnp.float32),
    mesh=_MESH,
    compiler_params=_SC_PARAMS,
    scratch_types=(
        pltpu.VMEM((18, CHUNK), jnp.int32),
        pltpu.VMEM((18, CHUNK), jnp.int32),
        pltpu.VMEM((18, CHUNK), jnp.float32),
        pltpu.VMEM((18, CHUNK), jnp.float32),
        pltpu.VMEM((2, CHUNK), jnp.int32),
        pltpu.VMEM((2, CHUNK, 128), jnp.float32),
        pltpu.VMEM_SHARED((NP, 128), jnp.float32),
        pltpu.SemaphoreType.DMA,
        pltpu.SemaphoreType.DMA,
    ),
)
def _sc_f4(src_hbm, dst_hbm, h_hbm, w_hbm, acc_out,
           sb, db, w0b, w1b, gidx, rowb, acc_sp, semg, sems):
  c = lax.axis_index("c")
  s = lax.axis_index("s")

  zero16 = jnp.zeros((16,), jnp.float32)
  for bb in range(2):
    @pl.loop(0, CHUNK)
    def _(i):
      for j in range(8):
        rowb[bb, i, pl.ds(j * 16, 16)] = zero16

  row0 = s * ROWS_PER_TILE
  for z in range(ROWS_PER_TILE // CHUNK):
    pltpu.sync_copy(rowb.at[0], acc_sp.at[pl.ds(row0 + z * CHUNK, CHUNK), :])
  plsc.subcore_barrier()

  pltpu.sync_copy(dst_hbm.at[pl.ds(0, 1), :], db.at[pl.ds(0, 1), :])
  pltpu.async_copy(rowb.at[0], acc_sp.at[db.at[0]], sems, add=True)
  pltpu.async_copy(rowb.at[1], acc_sp.at[db.at[0]], sems, add=True)

  coff = c * NP
  nblk = 9

  @pl.loop(0, nblk)
  def _(b):
    crow = s * 162 + b * 18
    pltpu.sync_copy(src_hbm.at[pl.ds(crow, 18), :], sb)
    pltpu.sync_copy(dst_hbm.at[pl.ds(crow, 18), :], db)
    pltpu.sync_copy(w_hbm.at[c, pl.ds(crow, 18), :], w0b)
    pltpu.sync_copy(w_hbm.at[c, pl.ds(NCHUNK + crow, 18), :], w1b)

    def scale(q, j):
      @pl.loop(0, CHUNK)
      def _(i):
        i16 = jnp.full((16,), i, jnp.int32)
        j16 = jnp.full((16,), j, jnp.int32)
        w0 = plsc.load_gather(w0b, [j16, i16])
        w1 = plsc.load_gather(w1b, [j16, i16])
        for t in range(8):
          v = rowb[q, i, pl.ds(t * 16, 16)]
          rowb[q, i, pl.ds(t * 16, 16)] = v * (w0 if t < 4 else w1)

    gdesc = [None, None]
    for j in range(18):
      pp = j & 1
      for t in range(8):
        gidx[pp, pl.ds(t * 16, 16)] = sb[j, pl.ds(t * 16, 16)] + coff
      pltpu.make_async_copy(rowb.at[pp], acc_sp.at[db.at[j]], sems).wait()
      gdesc[pp] = pltpu.async_copy(h_hbm.at[gidx.at[pp]], rowb.at[pp], semg)
      if j > 0:
        q = (j - 1) & 1
        gdesc[q].wait()
        scale(q, j - 1)
        pltpu.async_copy(rowb.at[q], acc_sp.at[db.at[j - 1]], sems, add=True)
    gdesc[1].wait()
    scale(1, 17)
    pltpu.async_copy(rowb.at[1], acc_sp.at[db.at[17]], sems, add=True)

  pltpu.make_async_copy(rowb.at[0], acc_sp.at[db.at[16]], sems).wait()
  pltpu.make_async_copy(rowb.at[1], acc_sp.at[db.at[17]], sems).wait()

  plsc.subcore_barrier()
  for z in range(ROWS_PER_TILE // CHUNK):
    rows = pl.ds(row0 + z * CHUNK, CHUNK)
    pltpu.sync_copy(acc_sp.at[rows, :], rowb.at[0])
    pltpu.sync_copy(rowb.at[0], acc_out.at[c, rows, :])


@functools.partial(
    pl.kernel,
    out_type=(jax.ShapeDtypeStruct((2, NP, 64), jnp.float32),
              jax.ShapeDtypeStruct((2, NP, 16), jnp.float32)),
    mesh=_MESH,
    compiler_params=_SC_PARAMS,
    scratch_types=(
        pltpu.VMEM((NP,), jnp.float32),
        pltpu.VMEM((NP,), jnp.float32),
        pltpu.VMEM((CHUNK,), jnp.int32),
        pltpu.VMEM((1, CHUNK), jnp.int32),
        pltpu.VMEM((CHUNK, 64), jnp.float32),
        pltpu.VMEM((CHUNK, 16), jnp.float32),
        pltpu.VMEM((CHUNK,), jnp.float32),
        pltpu.VMEM_SHARED((NP, 64), jnp.float32),
        pltpu.VMEM_SHARED((NP, 16), jnp.float32),
        pltpu.SemaphoreType.DMA,
    ),
)
def _sc_edge1(src_hbm, dst_hbm, h_hbm, als_hbm, ald_hbm, acc_out, den_out,
              als_l, ald_l, srcb, dstb, rowb, wvec, wb0, acc_sp, den_sp, sem):
  c = lax.axis_index("c")
  s = lax.axis_index("s")
  pltpu.sync_copy(als_hbm, als_l)
  pltpu.sync_copy(ald_hbm, ald_l)

  zero16 = jnp.zeros((16,), jnp.float32)

  @pl.loop(0, CHUNK)
  def _(i):
    for j in range(4):
      rowb[i, pl.ds(j * 16, 16)] = zero16
    wvec[i, :] = zero16

  row0 = s * ROWS_PER_TILE
  for z in range(ROWS_PER_TILE // CHUNK):
    pltpu.sync_copy(rowb, acc_sp.at[pl.ds(row0 + z * CHUNK, CHUNK), :])
    pltpu.sync_copy(wvec, den_sp.at[pl.ds(row0 + z * CHUNK, CHUNK), :])
  plsc.subcore_barrier()

  iota16 = lax.broadcasted_iota(jnp.int32, (16,), 0)
  per_tile = NCHUNK // 32

  @pl.loop(0, per_tile)
  def _(k):
    base = (c * (NCHUNK // 2) + s * per_tile + k) * CHUNK
    pltpu.sync_copy(src_hbm.at[pl.ds(base, CHUNK)], srcb)
    pltpu.sync_copy(dst_hbm.at[pl.ds(base, CHUNK)], dstb.at[0])
    cp = pltpu.async_copy(h_hbm.at[srcb], rowb, sem)
    for j in range(8):
      s16 = srcb[pl.ds(j * 16, 16)]
      d16 = dstb[0, pl.ds(j * 16, 16)]
      e16 = j * 16 + iota16
      av = plsc.load_gather(als_l, [s16])
      dv = plsc.load_gather(ald_l, [d16])
      t = av + dv
      w = jnp.exp(jnp.where(t >= 0, t, 0.2 * t))
      wb0[pl.ds(j * 16, 16)] = w
      plsc.store_scatter(wvec, [e16, jnp.zeros((16,), jnp.int32)], w)
    cp.wait()

    @pl.loop(0, CHUNK)
    def _(i):
      i16 = jnp.full((16,), i, jnp.int32)
      w0 = plsc.load_gather(wb0, [i16])
      for j in range(4):
        v = rowb[i, pl.ds(j * 16, 16)]
        rowb[i, pl.ds(j * 16, 16)] = v * w0

    pltpu.sync_copy(rowb, acc_sp.at[dstb.at[0]], add=True)
    pltpu.sync_copy(wvec, den_sp.at[dstb.at[0]], add=True)

  plsc.subcore_barrier()
  for z in range(ROWS_PER_TILE // CHUNK):
    rows = pl.ds(row0 + z * CHUNK, CHUNK)
    pltpu.sync_copy(acc_sp.at[rows, :], rowb)
    pltpu.sync_copy(rowb, acc_out.at[c, rows, :])
    pltpu.sync_copy(den_sp.at[rows, :], wvec)
    pltpu.sync_copy(wvec, den_out.at[c, rows, :])


def _block_diag_att(a):
  h, cdim = a.shape
  m = jnp.zeros((h * cdim, h), jnp.float32)
  for i in range(h):
    m = m.at[i * cdim:(i + 1) * cdim, i].set(a[i])
  return m


def kernel(x, edge_index, batch, W0, as0, ad0, b0, W1, as1, ad1, b1,
           W2, as2, ad2, b2, bnw0, bnb0, bnw1, bnb1, Wp1, bp1, Wp2, bp2):
  npad = EP - E2
  pad_idx = (N + (jnp.arange(npad, dtype=jnp.int32) % (NP - N))).astype(jnp.int32)
  loop_idx = jnp.arange(N, dtype=jnp.int32)
  src = jnp.concatenate([edge_index[0].astype(jnp.int32), loop_idx, pad_idx])
  dst = jnp.concatenate([edge_index[1].astype(jnp.int32), loop_idx, pad_idx])
  x_pad = jnp.pad(x, ((0, NP - N), (0, 0)))
  batch_col = jnp.pad(batch.astype(jnp.int32), (0, NP - N),
                      constant_values=G).reshape(NP, 1)

  As0 = _block_diag_att(as0)
  Ad0 = _block_diag_att(ad0)
  As1 = _block_diag_att(as1)
  Ad1 = _block_diag_att(ad1)
  As2 = as2.reshape(64, 1)
  Ad2 = ad2.reshape(64, 1)

  inv = jnp.float32(1.0 / jnp.sqrt(1.0 + 1e-5))
  b0r = b0.reshape(1, 256)
  sc0 = (bnw0 * inv).reshape(1, 256)
  sh0 = bnb0.reshape(1, 256)
  b1r = b1.reshape(1, 256)
  sc1 = (bnw1 * inv).reshape(1, 256)
  sh1 = bnb1.reshape(1, 256)
  b2r = b2.reshape(1, 64)

  hp0, als0v, ald0v = _tc_a0(x_pad, W0, As0, Ad0)
  h0flat = hp0.reshape(2 * NP, 128)
  als0p = als0v.reshape(NP, 2, 2).transpose(1, 0, 2).reshape(4 * NP)
  ald0p = ald0v.reshape(NP, 2, 2).transpose(1, 0, 2).reshape(4 * NP)
  src2d = src.reshape(NCHUNK, CHUNK)
  dst2d = dst.reshape(NCHUNK, CHUNK)
  w0e, den0 = _sc_w4(src, dst, als0p, ald0p)
  acc0 = _sc_f4(src2d, dst2d, h0flat, w0e)

  hp1, als1v, ald1v = _tc_mid(acc0, den0, b0r, sc0, sh0, W1, As1, Ad1, 4)
  h1flat = hp1.reshape(2 * NP, 128)
  als1p = als1v.reshape(NP, 2, 2).transpose(1, 0, 2).reshape(4 * NP)
  ald1p = ald1v.reshape(NP, 2, 2).transpose(1, 0, 2).reshape(4 * NP)
  w1e, den1 = _sc_w4(src, dst, als1p, ald1p)
  acc1 = _sc_f4(src2d, dst2d, h1flat, w1e)

  h2, als2v, ald2v = _tc_mid(acc1, den1, b1r, sc1, sh1, W2, As2, Ad2, 1)
  acc2, den2 = _sc_edge1(src, dst, h2, als2v.reshape(NP), ald2v.reshape(NP))

  return _tc_final(acc2, den2, b2r, batch_col, Wp1, bp1.reshape(1, 64),
                   Wp2, bp2.reshape(1, 64))


# speedup vs baseline: 55.3653x; 1.3237x over previous
import functools

import jax
import jax.numpy as jnp
from jax import lax
from jax.experimental import pallas as pl
from jax.experimental.pallas import tpu as pltpu
from jax.experimental.pallas import tpu_sc as plsc

N = 10000
NP = 10240
E = 320000
E2 = E + N
CHUNK = 128
NCHUNK = 2592
EP = NCHUNK * CHUNK
ROWS_PER_TILE = NP // 16
G = 64
NEG_INF = float("-inf")


def _tc_a0_body(x_ref, w_ref, as_ref, ad_ref, hp_ref, als_ref, ald_ref):
  h = jnp.dot(x_ref[...], w_ref[...], preferred_element_type=jnp.float32)
  hp_ref[0] = h[:, :128]
  hp_ref[1] = h[:, 128:]
  als_ref[...] = jnp.dot(h, as_ref[...], preferred_element_type=jnp.float32)
  ald_ref[...] = jnp.dot(h, ad_ref[...], preferred_element_type=jnp.float32)


def _tc_a0(x_pad, W0, As, Ad):
  blk = NP // 8
  return pl.pallas_call(
      _tc_a0_body,
      grid=(8,),
      in_specs=[
          pl.BlockSpec((blk, 128), lambda i: (i, 0)),
          pl.BlockSpec((128, 256), lambda i: (0, 0)),
          pl.BlockSpec((256, 4), lambda i: (0, 0)),
          pl.BlockSpec((256, 4), lambda i: (0, 0)),
      ],
      out_specs=[
          pl.BlockSpec((2, blk, 128), lambda i: (0, i, 0)),
          pl.BlockSpec((blk, 4), lambda i: (i, 0)),
          pl.BlockSpec((blk, 4), lambda i: (i, 0)),
      ],
      out_shape=[
          jax.ShapeDtypeStruct((2, NP, 128), jnp.float32),
          jax.ShapeDtypeStruct((NP, 4), jnp.float32),
          jax.ShapeDtypeStruct((NP, 4), jnp.float32),
      ],
  )(x_pad, W0, As, Ad)


def _tc_mid_body(acc_ref, den_ref, b_ref, sc_ref, sh_ref, w_ref, as_ref,
                 ad_ref, hp_ref, als_ref, ald_ref, *, nheads_next):
  a0 = acc_ref[0]
  a1 = acc_ref[1]
  d0 = den_ref[0]
  d1 = den_ref[1]
  eps = jnp.float32(1e-16)
  p0 = a0[:, :64] / (d0[:, 0:1] + eps)
  p1 = a0[:, 64:] / (d0[:, 1:2] + eps)
  p2 = a1[:, :64] / (d1[:, 0:1] + eps)
  p3 = a1[:, 64:] / (d1[:, 1:2] + eps)
  xx = jnp.concatenate([p0, p1, p2, p3], axis=1) + b_ref[...]
  xx = xx * sc_ref[...] + sh_ref[...]
  xx = jnp.where(xx > 0, xx, jnp.exp(xx) - 1.0)
  h = jnp.dot(xx, w_ref[...], preferred_element_type=jnp.float32)
  if nheads_next == 4:
    hp_ref[0] = h[:, :128]
    hp_ref[1] = h[:, 128:]
  else:
    hp_ref[...] = h
  als_ref[...] = jnp.dot(h, as_ref[...], preferred_element_type=jnp.float32)
  ald_ref[...] = jnp.dot(h, ad_ref[...], preferred_element_type=jnp.float32)


def _tc_mid(acc, den, b_row, sc_row, sh_row, W, As, Ad, nheads_next):
  blk = NP // 8
  dout = W.shape[1]
  nh = nheads_next
  if nh == 4:
    hp_spec = pl.BlockSpec((2, blk, 128), lambda i: (0, i, 0))
    hp_shape = jax.ShapeDtypeStruct((2, NP, 128), jnp.float32)
  else:
    hp_spec = pl.BlockSpec((blk, 64), lambda i: (i, 0))
    hp_shape = jax.ShapeDtypeStruct((NP, 64), jnp.float32)
  return pl.pallas_call(
      functools.partial(_tc_mid_body, nheads_next=nh),
      grid=(8,),
      in_specs=[
          pl.BlockSpec((2, blk, 128), lambda i: (0, i, 0)),
          pl.BlockSpec((2, blk, 16), lambda i: (0, i, 0)),
          pl.BlockSpec((1, 256), lambda i: (0, 0)),
          pl.BlockSpec((1, 256), lambda i: (0, 0)),
          pl.BlockSpec((1, 256), lambda i: (0, 0)),
          pl.BlockSpec((256, dout), lambda i: (0, 0)),
          pl.BlockSpec((dout, nh), lambda i: (0, 0)),
          pl.BlockSpec((dout, nh), lambda i: (0, 0)),
      ],
      out_specs=[
          hp_spec,
          pl.BlockSpec((blk, nh), lambda i: (i, 0)),
          pl.BlockSpec((blk, nh), lambda i: (i, 0)),
      ],
      out_shape=[
          hp_shape,
          jax.ShapeDtypeStruct((NP, nh), jnp.float32),
          jax.ShapeDtypeStruct((NP, nh), jnp.float32),
      ],
  )(acc, den, b_row, sc_row, sh_row, W, As, Ad)


def _tc_final_body(acc_ref, den_ref, b2_ref, batch_ref, wp1_ref, bp1_ref,
                   wp2_ref, bp2_ref, out_ref, xm_ref):
  eps = jnp.float32(1e-16)
  den = den_ref[0][:, 0:1] + den_ref[1][:, 0:1] + eps
  x3 = (acc_ref[0] + acc_ref[1]) / den + b2_ref[...]
  bcol = batch_ref[...]
  gids = lax.broadcasted_iota(jnp.int32, (1, G), 1)
  S = (bcol == gids).astype(jnp.float32)
  counts = jnp.sum(S, axis=0)
  xsum = lax.dot_general(S, x3, (((0,), (0,)), ((), ())),
                         preferred_element_type=jnp.float32)
  xmean = xsum / jnp.maximum(counts, 1.0)[:, None]
  def _seg_max(g, _):
    mg = jnp.where(bcol == g, x3, NEG_INF)
    xm_ref[pl.ds(g, 1), :] = jnp.max(mg, axis=0)[None, :]
    return 0
  lax.fori_loop(0, G, _seg_max, 0)
  xmax = jnp.where(counts[:, None] > 0, xm_ref[...], 0.0)
  xcat = jnp.concatenate([xmax, xmean], axis=1)
  h1 = jnp.dot(xcat, wp1_ref[...], preferred_element_type=jnp.float32)
  h1 = jnp.maximum(h1 + bp1_ref[...], 0.0)
  out_ref[...] = jnp.dot(h1, wp2_ref[...],
                         preferred_element_type=jnp.float32) + bp2_ref[...]


def _tc_final(acc2, den2, b2_row, batch_col, Wp1, bp1_row, Wp2, bp2_row):
  return pl.pallas_call(
      _tc_final_body,
      out_shape=jax.ShapeDtypeStruct((G, 64), jnp.float32),
      scratch_shapes=[pltpu.VMEM((G, 64), jnp.float32)],
  )(acc2, den2, b2_row, batch_col, Wp1, bp1_row, Wp2, bp2_row)


_MESH = plsc.VectorSubcoreMesh(core_axis_name="c", subcore_axis_name="s")
_SC_PARAMS = pltpu.CompilerParams(needs_layout_passes=False,
                                  use_tc_tiling_on_sc=False)


@functools.partial(
    pl.kernel,
    out_type=(jax.ShapeDtypeStruct((2, 2 * NCHUNK, CHUNK), jnp.float32),
              jax.ShapeDtypeStruct((2, NP, 16), jnp.float32)),
    mesh=_MESH,
    compiler_params=_SC_PARAMS,
    scratch_types=(
        pltpu.VMEM((2 * NP,), jnp.float32),
        pltpu.VMEM((2 * NP,), jnp.float32),
        pltpu.VMEM((18, CHUNK), jnp.int32),
        pltpu.VMEM((18, CHUNK), jnp.int32),
        pltpu.VMEM((18, CHUNK), jnp.float32),
        pltpu.VMEM((18, CHUNK), jnp.float32),
        pltpu.VMEM((2, CHUNK, 16), jnp.float32),
        pltpu.VMEM_SHARED((NP, 16), jnp.float32),
        pltpu.SemaphoreType.DMA,
    ),
)
def _sc_w4(src_hbm, dst_hbm, als_hbm, ald_hbm, w_out, den_out,
           als_l, ald_l, sb, db, w0o, w1o, wvec, den_sp, semd):
  c = lax.axis_index("c")
  s = lax.axis_index("s")
  pltpu.sync_copy(als_hbm.at[pl.ds(c * 2 * NP, 2 * NP)], als_l)
  pltpu.sync_copy(ald_hbm.at[pl.ds(c * 2 * NP, 2 * NP)], ald_l)

  zero16 = jnp.zeros((16,), jnp.float32)
  for bb in range(2):
    @pl.loop(0, CHUNK)
    def _(i):
      wvec[bb, i, :] = zero16

  row0 = s * ROWS_PER_TILE
  for z in range(ROWS_PER_TILE // CHUNK):
    pltpu.sync_copy(wvec.at[0], den_sp.at[pl.ds(row0 + z * CHUNK, CHUNK), :])
  plsc.subcore_barrier()

  pltpu.sync_copy(dst_hbm.at[pl.ds(0, 1), :], db.at[pl.ds(0, 1), :])
  pltpu.async_copy(wvec.at[0], den_sp.at[db.at[0]], semd, add=True)
  pltpu.async_copy(wvec.at[1], den_sp.at[db.at[0]], semd, add=True)

  iota16 = lax.broadcasted_iota(jnp.int32, (16,), 0)

  @pl.loop(0, 9)
  def _(b):
    crow = s * 162 + b * 18
    pltpu.sync_copy(src_hbm.at[pl.ds(crow, 18), :], sb)
    pltpu.sync_copy(dst_hbm.at[pl.ds(crow, 18), :], db)
    for j in range(18):
      pp = j & 1
      pltpu.make_async_copy(wvec.at[pp], den_sp.at[db.at[j]], semd).wait()
      for tt in range(8):
        s16 = sb[j, pl.ds(tt * 16, 16)]
        d16 = db[j, pl.ds(tt * 16, 16)]
        e16 = tt * 16 + iota16
        for hd in range(2):
          av = plsc.load_gather(als_l, [s16 * 2 + hd])
          dv = plsc.load_gather(ald_l, [d16 * 2 + hd])
          tv = av + dv
          w = jnp.exp(jnp.where(tv >= 0, tv, 0.2 * tv))
          if hd == 0:
            w0o[j, pl.ds(tt * 16, 16)] = w
          else:
            w1o[j, pl.ds(tt * 16, 16)] = w
          plsc.store_scatter(wvec.at[pp],
                             [e16, jnp.full((16,), hd, jnp.int32)], w)
      pltpu.async_copy(wvec.at[pp], den_sp.at[db.at[j]], semd, add=True)
    pltpu.sync_copy(w0o, w_out.at[c, pl.ds(crow, 18), :])
    pltpu.sync_copy(w1o, w_out.at[c, pl.ds(NCHUNK + crow, 18), :])

  pltpu.make_async_copy(wvec.at[0], den_sp.at[db.at[16]], semd).wait()
  pltpu.make_async_copy(wvec.at[1], den_sp.at[db.at[17]], semd).wait()

  plsc.subcore_barrier()
  for z in range(ROWS_PER_TILE // CHUNK):
    rows = pl.ds(row0 + z * CHUNK, CHUNK)
    pltpu.sync_copy(den_sp.at[rows, :], wvec.at[0])
    pltpu.sync_copy(wvec.at[0], den_out.at[c, rows, :])


@functools.partial(
    pl.kernel,
    out_type=jax.ShapeDtypeStruct((2, NP, 128), jnp.float32),
    mesh=_MESH,
    compiler_params=_SC_PARAMS,
    scratch_types=(
        pltpu.VMEM((18, CHUNK), jnp.int32),
        pltpu.VMEM((18, CHUNK), jnp.int32),
        pltpu.VMEM((18, CHUNK), jnp.float32),
        pltpu.VMEM((18, CHUNK), jnp.float32),
        pltpu.VMEM((2, CHUNK), jnp.int32),
        pltpu.VMEM((2, CHUNK, 128), jnp.float32),
        pltpu.VMEM_SHARED((NP, 128), jnp.float32),
        pltpu.SemaphoreType.DMA,
        pltpu.SemaphoreType.DMA,
    ),
)
def _sc_f4(src_hbm, dst_hbm, h_hbm, w_hbm, acc_out,
           sb, db, w0b, w1b, gidx, rowb, acc_sp, semg, sems):
  c = lax.axis_index("c")
  s = lax.axis_index("s")

  zero16 = jnp.zeros((16,), jnp.float32)
  for bb in range(2):
    @pl.loop(0, CHUNK)
    def _(i):
      for j in range(8):
        rowb[bb, i, pl.ds(j * 16, 16)] = zero16

  row0 = s * ROWS_PER_TILE
  for z in range(ROWS_PER_TILE // CHUNK):
    pltpu.sync_copy(rowb.at[0], acc_sp.at[pl.ds(row0 + z * CHUNK, CHUNK), :])
  plsc.subcore_barrier()

  pltpu.sync_copy(dst_hbm.at[pl.ds(0, 1), :], db.at[pl.ds(0, 1), :])
  pltpu.async_copy(rowb.at[0], acc_sp.at[db.at[0]], sems, add=True)
  pltpu.async_copy(rowb.at[1], acc_sp.at[db.at[0]], sems, add=True)

  coff = c * NP
  nblk = 9

  @pl.loop(0, nblk)
  def _(b):
    crow = s * 162 + b * 18
    pltpu.sync_copy(src_hbm.at[pl.ds(crow, 18), :], sb)
    pltpu.sync_copy(dst_hbm.at[pl.ds(crow, 18), :], db)
    pltpu.sync_copy(w_hbm.at[c, pl.ds(crow, 18), :], w0b)
    pltpu.sync_copy(w_hbm.at[c, pl.ds(NCHUNK + crow, 18), :], w1b)

    def scale(q, j):
      @pl.loop(0, CHUNK)
      def _(i):
        i16 = jnp.full((16,), i, jnp.int32)
        j16 = jnp.full((16,), j, jnp.int32)
        w0 = plsc.load_gather(w0b, [j16, i16])
        w1 = plsc.load_gather(w1b, [j16, i16])
        for t in range(8):
          v = rowb[q, i, pl.ds(t * 16, 16)]
          rowb[q, i, pl.ds(t * 16, 16)] = v * (w0 if t < 4 else w1)

    gdesc = [None, None]
    for j in range(18):
      pp = j & 1
      for t in range(8):
        gidx[pp, pl.ds(t * 16, 16)] = sb[j, pl.ds(t * 16, 16)] + coff
      pltpu.make_async_copy(rowb.at[pp], acc_sp.at[db.at[j]], sems).wait()
      gdesc[pp] = pltpu.async_copy(h_hbm.at[gidx.at[pp]], rowb.at[pp], semg)
      if j > 0:
        q = (j - 1) & 1
        gdesc[q].wait()
        scale(q, j - 1)
        pltpu.async_copy(rowb.at[q], acc_sp.at[db.at[j - 1]], sems, add=True)
    gdesc[1].wait()
    scale(1, 17)
    pltpu.async_copy(rowb.at[1], acc_sp.at[db.at[17]], sems, add=True)

  pltpu.make_async_copy(rowb.at[0], acc_sp.at[db.at[16]], sems).wait()
  pltpu.make_async_copy(rowb.at[1], acc_sp.at[db.at[17]], sems).wait()

  plsc.subcore_barrier()
  for z in range(ROWS_PER_TILE // CHUNK):
    rows = pl.ds(row0 + z * CHUNK, CHUNK)
    pltpu.sync_copy(acc_sp.at[rows, :], rowb.at[0])
    pltpu.sync_copy(rowb.at[0], acc_out.at[c, rows, :])


@functools.partial(
    pl.kernel,
    out_type=(jax.ShapeDtypeStruct((2, NP, 64), jnp.float32),
              jax.ShapeDtypeStruct((2, NP, 16), jnp.float32)),
    mesh=_MESH,
    compiler_params=_SC_PARAMS,
    scratch_types=(
        pltpu.VMEM((NP,), jnp.float32),
        pltpu.VMEM((NP,), jnp.float32),
        pltpu.VMEM((27, CHUNK), jnp.int32),
        pltpu.VMEM((27, CHUNK), jnp.int32),
        pltpu.VMEM((2, CHUNK), jnp.float32),
        pltpu.VMEM((2, CHUNK, 16), jnp.float32),
        pltpu.VMEM((2, CHUNK, 64), jnp.float32),
        pltpu.VMEM_SHARED((NP, 64), jnp.float32),
        pltpu.VMEM_SHARED((NP, 16), jnp.float32),
        pltpu.SemaphoreType.DMA,
        pltpu.SemaphoreType.DMA,
        pltpu.SemaphoreType.DMA,
    ),
)
def _sc_edge1(src_hbm, dst_hbm, h_hbm, als_hbm, ald_hbm, acc_out, den_out,
              als_l, ald_l, sb, db, wb, wvec, rowb, acc_sp, den_sp,
              semg, semr, semd):
  c = lax.axis_index("c")
  s = lax.axis_index("s")
  pltpu.sync_copy(als_hbm, als_l)
  pltpu.sync_copy(ald_hbm, ald_l)

  zero16 = jnp.zeros((16,), jnp.float32)
  for bb in range(2):
    @pl.loop(0, CHUNK)
    def _(i):
      for j in range(4):
        rowb[bb, i, pl.ds(j * 16, 16)] = zero16
      wvec[bb, i, :] = zero16

  row0 = s * ROWS_PER_TILE
  for z in range(ROWS_PER_TILE // CHUNK):
    pltpu.sync_copy(rowb.at[0], acc_sp.at[pl.ds(row0 + z * CHUNK, CHUNK), :])
    pltpu.sync_copy(wvec.at[0], den_sp.at[pl.ds(row0 + z * CHUNK, CHUNK), :])
  plsc.subcore_barrier()

  pltpu.sync_copy(dst_hbm.at[pl.ds(0, 1), :], db.at[pl.ds(0, 1), :])
  for bb in range(2):
    pltpu.async_copy(rowb.at[bb], acc_sp.at[db.at[0]], semr, add=True)
    pltpu.async_copy(wvec.at[bb], den_sp.at[db.at[0]], semd, add=True)

  iota16 = lax.broadcasted_iota(jnp.int32, (16,), 0)
  z16 = jnp.zeros((16,), jnp.int32)

  @pl.loop(0, 3)
  def _(b):
    crow = c * (NCHUNK // 2) + s * 81 + b * 27
    pltpu.sync_copy(src_hbm.at[pl.ds(crow, 27), :], sb)
    pltpu.sync_copy(dst_hbm.at[pl.ds(crow, 27), :], db)

    def scale(q, j):
      @pl.loop(0, CHUNK)
      def _(i):
        i16 = jnp.full((16,), i, jnp.int32)
        w0 = plsc.load_gather(wb.at[q], [i16])
        for tt in range(4):
          v = rowb[q, i, pl.ds(tt * 16, 16)]
          rowb[q, i, pl.ds(tt * 16, 16)] = v * w0

    gdesc = [None, None]
    for j in range(27):
      pp = j & 1
      pltpu.make_async_copy(rowb.at[pp], acc_sp.at[db.at[j]], semr).wait()
      gdesc[pp] = pltpu.async_copy(h_hbm.at[sb.at[j]], rowb.at[pp], semg)
      pltpu.make_async_copy(wvec.at[pp], den_sp.at[db.at[j]], semd).wait()
      for tt in range(8):
        s16 = sb[j, pl.ds(tt * 16, 16)]
        d16 = db[j, pl.ds(tt * 16, 16)]
        e16 = tt * 16 + iota16
        av = plsc.load_gather(als_l, [s16])
        dv = plsc.load_gather(ald_l, [d16])
        tv = av + dv
        w = jnp.exp(jnp.where(tv >= 0, tv, 0.2 * tv))
        wb[pp, pl.ds(tt * 16, 16)] = w
        plsc.store_scatter(wvec.at[pp], [e16, z16], w)
      pltpu.async_copy(wvec.at[pp], den_sp.at[db.at[j]], semd, add=True)
      if j > 0:
        q = (j - 1) & 1
        gdesc[q].wait()
        scale(q, j - 1)
        pltpu.async_copy(rowb.at[q], acc_sp.at[db.at[j - 1]], semr, add=True)
    gdesc[0].wait()
    scale(0, 26)
    pltpu.async_copy(rowb.at[0], acc_sp.at[db.at[26]], semr, add=True)

  pltpu.make_async_copy(rowb.at[0], acc_sp.at[db.at[26]], semr).wait()
  pltpu.make_async_copy(rowb.at[1], acc_sp.at[db.at[25]], semr).wait()
  pltpu.make_async_copy(wvec.at[0], den_sp.at[db.at[26]], semd).wait()
  pltpu.make_async_copy(wvec.at[1], den_sp.at[db.at[25]], semd).wait()

  plsc.subcore_barrier()
  for z in range(ROWS_PER_TILE // CHUNK):
    rows = pl.ds(row0 + z * CHUNK, CHUNK)
    pltpu.sync_copy(acc_sp.at[rows, :], rowb.at[0])
    pltpu.sync_copy(rowb.at[0], acc_out.at[c, rows, :])
    pltpu.sync_copy(den_sp.at[rows, :], wvec.at[0])
    pltpu.sync_copy(wvec.at[0], den_out.at[c, rows, :])


def _block_diag_att(a):
  h, cdim = a.shape
  m = jnp.zeros((h * cdim, h), jnp.float32)
  for i in range(h):
    m = m.at[i * cdim:(i + 1) * cdim, i].set(a[i])
  return m


def kernel(x, edge_index, batch, W0, as0, ad0, b0, W1, as1, ad1, b1,
           W2, as2, ad2, b2, bnw0, bnb0, bnw1, bnb1, Wp1, bp1, Wp2, bp2):
  npad = EP - E2
  pad_idx = (N + (jnp.arange(npad, dtype=jnp.int32) % (NP - N))).astype(jnp.int32)
  loop_idx = jnp.arange(N, dtype=jnp.int32)
  src = jnp.concatenate([edge_index[0].astype(jnp.int32), loop_idx, pad_idx])
  dst = jnp.concatenate([edge_index[1].astype(jnp.int32), loop_idx, pad_idx])
  x_pad = jnp.pad(x, ((0, NP - N), (0, 0)))
  batch_col = jnp.pad(batch.astype(jnp.int32), (0, NP - N),
                      constant_values=G).reshape(NP, 1)

  As0 = _block_diag_att(as0)
  Ad0 = _block_diag_att(ad0)
  As1 = _block_diag_att(as1)
  Ad1 = _block_diag_att(ad1)
  As2 = as2.reshape(64, 1)
  Ad2 = ad2.reshape(64, 1)

  inv = jnp.float32(1.0 / jnp.sqrt(1.0 + 1e-5))
  b0r = b0.reshape(1, 256)
  sc0 = (bnw0 * inv).reshape(1, 256)
  sh0 = bnb0.reshape(1, 256)
  b1r = b1.reshape(1, 256)
  sc1 = (bnw1 * inv).reshape(1, 256)
  sh1 = bnb1.reshape(1, 256)
  b2r = b2.reshape(1, 64)

  hp0, als0v, ald0v = _tc_a0(x_pad, W0, As0, Ad0)
  h0flat = hp0.reshape(2 * NP, 128)
  als0p = als0v.reshape(NP, 2, 2).transpose(1, 0, 2).reshape(4 * NP)
  ald0p = ald0v.reshape(NP, 2, 2).transpose(1, 0, 2).reshape(4 * NP)
  src2d = src.reshape(NCHUNK, CHUNK)
  dst2d = dst.reshape(NCHUNK, CHUNK)
  w0e, den0 = _sc_w4(src2d, dst2d, als0p, ald0p)
  acc0 = _sc_f4(src2d, dst2d, h0flat, w0e)

  hp1, als1v, ald1v = _tc_mid(acc0, den0, b0r, sc0, sh0, W1, As1, Ad1, 4)
  h1flat = hp1.reshape(2 * NP, 128)
  als1p = als1v.reshape(NP, 2, 2).transpose(1, 0, 2).reshape(4 * NP)
  ald1p = ald1v.reshape(NP, 2, 2).transpose(1, 0, 2).reshape(4 * NP)
  w1e, den1 = _sc_w4(src2d, dst2d, als1p, ald1p)
  acc1 = _sc_f4(src2d, dst2d, h1flat, w1e)

  h2, als2v, ald2v = _tc_mid(acc1, den1, b1r, sc1, sh1, W2, As2, Ad2, 1)
  acc2, den2 = _sc_edge1(src2d, dst2d, h2, als2v.reshape(NP), ald2v.reshape(NP))

  return _tc_final(acc2, den2, b2r, batch_col, Wp1, bp1.reshape(1, 64),
                   Wp2, bp2.reshape(1, 64))
